# Initial kernel scaffold; baseline (speedup 1.0000x reference)
import jax
import jax.numpy as jnp
from jax.experimental import pallas as pl


def kernel(x, W1, b1, W2, b2, S1, s1, S2, s2, We, be):
    raise NotImplementedError("write your pallas kernel here")


# speedup vs baseline: 1.3157x; 1.3157x over previous
import functools

import jax
import jax.numpy as jnp
from jax.experimental import pallas as pl

TOP_K = 2
EXPLORATION_BONUS = 0.1


def _body(x_ref, w1_ref, b1_ref, w2_ref, b2_ref, s1_ref, s1b_ref,
          s2_ref, s2b_ref, wem_ref, bev_ref, pred_ref, rw_ref, *, n_experts):
    xt = x_ref[...]
    h = jnp.maximum(
        jnp.dot(xt, w1_ref[...], preferred_element_type=jnp.float32)
        + b1_ref[...], 0.0)
    ctx = jnp.tanh(
        jnp.dot(h, w2_ref[...], preferred_element_type=jnp.float32)
        + b2_ref[...])
    sh = jnp.maximum(
        jnp.dot(ctx, s1_ref[...], preferred_element_type=jnp.float32)
        + s1b_ref[...], 0.0)
    scores = (jnp.dot(sh, s2_ref[...], preferred_element_type=jnp.float32)
              + s2b_ref[...] + EXPLORATION_BONUS)
    preds = (jnp.dot(xt, wem_ref[...], preferred_element_type=jnp.float32)
             + bev_ref[...])

    eidx = jax.lax.broadcasted_iota(jnp.int32, scores.shape, 1)
    m1 = jnp.max(scores, axis=1, keepdims=True)
    i1 = jnp.min(jnp.where(scores == m1, eidx, n_experts), axis=1,
                 keepdims=True)
    masked = jnp.where(eidx == i1, -jnp.inf, scores)
    m2 = jnp.max(masked, axis=1, keepdims=True)
    i2 = jnp.min(jnp.where(masked == m2, eidx, n_experts), axis=1,
                 keepdims=True)

    e2 = jnp.exp(m2 - m1)
    denom = 1.0 + e2
    w1v = 1.0 / denom
    w2v = e2 / denom

    sel = jnp.where(eidx == i1, w1v, 0.0) + jnp.where(eidx == i2, w2v, 0.0)
    pred_ref[...] = jnp.sum(sel * preds, axis=1, keepdims=True)
    rw_ref[...] = jnp.concatenate([w1v, w2v], axis=1)


def kernel(x, W1, b1, W2, b2, S1, s1, S2, s2, We, be):
    n, d = x.shape
    e = S2.shape[1]
    hid1 = W1.shape[1]
    ctxd = W2.shape[1]
    hid2 = S1.shape[1]

    wem = We[:, :, 0].T
    bev = be[:, 0].reshape(1, e)
    b1r = b1.reshape(1, hid1)
    b2r = b2.reshape(1, ctxd)
    s1r = s1.reshape(1, hid2)
    s2r = s2.reshape(1, e)

    tile = 512
    grid = n // tile
    const = lambda i: (0, 0)

    preds, rw = pl.pallas_call(
        functools.partial(_body, n_experts=e),
        grid=(grid,),
        in_specs=[
            pl.BlockSpec((tile, d), lambda i: (i, 0)),
            pl.BlockSpec((d, hid1), const),
            pl.BlockSpec((1, hid1), const),
            pl.BlockSpec((hid1, ctxd), const),
            pl.BlockSpec((1, ctxd), const),
            pl.BlockSpec((ctxd, hid2), const),
            pl.BlockSpec((1, hid2), const),
            pl.BlockSpec((hid2, e), const),
            pl.BlockSpec((1, e), const),
            pl.BlockSpec((d, e), const),
            pl.BlockSpec((1, e), const),
        ],
        out_specs=[
            pl.BlockSpec((tile, 1), lambda i: (i, 0)),
            pl.BlockSpec((tile, TOP_K), lambda i: (i, 0)),
        ],
        out_shape=[
            jax.ShapeDtypeStruct((n, 1), jnp.float32),
            jax.ShapeDtypeStruct((n, TOP_K), jnp.float32),
        ],
    )(x, W1, b1r, W2, b2r, S1, s1r, S2, s2r, wem, bev)
    return (preds, rw)


# speedup vs baseline: 1.4049x; 1.0678x over previous
import functools

import jax
import jax.numpy as jnp
from jax.experimental import pallas as pl

TOP_K = 2
EXPLORATION_BONUS = 0.1


def _body(x_ref, w1_ref, b1_ref, w2_ref, b2_ref, s1_ref, s1b_ref,
          s2_ref, s2b_ref, wem_ref, bev_ref, pred_ref, rw_ref, *, n_experts):
    xt = x_ref[...]
    xb = xt.astype(jnp.bfloat16)
    h = jnp.maximum(
        jnp.dot(xt, w1_ref[...], preferred_element_type=jnp.float32)
        + b1_ref[...], 0.0)
    ctx = jnp.tanh(
        jnp.dot(h, w2_ref[...], preferred_element_type=jnp.float32)
        + b2_ref[...])
    sh = jnp.maximum(
        jnp.dot(ctx, s1_ref[...], preferred_element_type=jnp.float32)
        + s1b_ref[...], 0.0)
    scores = (jnp.dot(sh, s2_ref[...], preferred_element_type=jnp.float32)
              + s2b_ref[...] + EXPLORATION_BONUS)
    preds = (jnp.dot(xb, wem_ref[...], preferred_element_type=jnp.float32)
             + bev_ref[...])

    eidx = jax.lax.broadcasted_iota(jnp.int32, scores.shape, 1).astype(
        jnp.float32)
    m1 = jnp.max(scores, axis=1, keepdims=True)
    i1 = jnp.min(jnp.where(scores == m1, eidx, float(n_experts)), axis=1,
                 keepdims=True)
    masked = jnp.where(eidx == i1, -jnp.inf, scores)
    m2 = jnp.max(masked, axis=1, keepdims=True)
    i2 = jnp.min(jnp.where(masked == m2, eidx, float(n_experts)), axis=1,
                 keepdims=True)

    e2 = jnp.exp(m2 - m1)
    denom = 1.0 + e2
    w1v = 1.0 / denom
    w2v = e2 / denom

    sel = jnp.where(eidx == i1, w1v, 0.0) + jnp.where(eidx == i2, w2v, 0.0)
    pred_ref[...] = jnp.sum(sel * preds, axis=1, keepdims=True)
    rw_ref[...] = jnp.concatenate([w1v, w2v], axis=1)


def kernel(x, W1, b1, W2, b2, S1, s1, S2, s2, We, be):
    n, d = x.shape
    e = S2.shape[1]
    hid1 = W1.shape[1]
    ctxd = W2.shape[1]
    hid2 = S1.shape[1]

    wem = We[:, :, 0].T.astype(jnp.bfloat16)
    bev = be[:, 0].reshape(1, e)
    b1r = b1.reshape(1, hid1)
    b2r = b2.reshape(1, ctxd)
    s1r = s1.reshape(1, hid2)
    s2r = s2.reshape(1, e)

    tile = 512
    grid = n // tile
    const = lambda i: (0, 0)

    preds, rw = pl.pallas_call(
        functools.partial(_body, n_experts=e),
        grid=(grid,),
        in_specs=[
            pl.BlockSpec((tile, d), lambda i: (i, 0)),
            pl.BlockSpec((d, hid1), const),
            pl.BlockSpec((1, hid1), const),
            pl.BlockSpec((hid1, ctxd), const),
            pl.BlockSpec((1, ctxd), const),
            pl.BlockSpec((ctxd, hid2), const),
            pl.BlockSpec((1, hid2), const),
            pl.BlockSpec((hid2, e), const),
            pl.BlockSpec((1, e), const),
            pl.BlockSpec((d, e), const),
            pl.BlockSpec((1, e), const),
        ],
        out_specs=[
            pl.BlockSpec((tile, 1), lambda i: (i, 0)),
            pl.BlockSpec((tile, TOP_K), lambda i: (i, 0)),
        ],
        out_shape=[
            jax.ShapeDtypeStruct((n, 1), jnp.float32),
            jax.ShapeDtypeStruct((n, TOP_K), jnp.float32),
        ],
    )(x, W1, b1r, W2, b2r, S1, s1r, S2, s2r, wem, bev)
    return (preds, rw)
